# baseline (speedup 1.0000x reference)
import jax
import jax.numpy as jnp
from jax.experimental import pallas as pl

_RR = 512


def _make_kernel(T, D):
    U = _RR // D

    def _add_kernel(x_ref, pe_ref, o_ref):
        i = pl.program_id(0)
        peT = pe_ref[...]
        rep = jnp.broadcast_to(peT[None], (U, D, T)).reshape(_RR, T)
        sub = jax.lax.broadcasted_iota(jnp.int32, (_RR, T), 0)
        lane = jax.lax.broadcasted_iota(jnp.int32, (_RR, T), 1)
        mask = lane == (U * i + sub // D)
        pe_col = jnp.sum(jnp.where(mask, rep, 0.0), axis=1, keepdims=True)
        o_ref[0] = x_ref[0] + pe_col

    return _add_kernel


def kernel(x, pos_emb):
    B, T, D = x.shape
    N = T * D
    G = N // _RR
    xt = x.transpose(1, 2, 0).reshape(G, _RR, B)
    out = pl.pallas_call(
        _make_kernel(T, D),
        grid=(G,),
        in_specs=[
            pl.BlockSpec((1, _RR, B), lambda i: (i, 0, 0)),
            pl.BlockSpec((D, T), lambda i: (0, 0)),
        ],
        out_specs=pl.BlockSpec((1, _RR, B), lambda i: (i, 0, 0)),
        out_shape=jax.ShapeDtypeStruct((G, _RR, B), x.dtype),
    )(xt, pos_emb.T)
    return out.reshape(T, D, B).transpose(2, 0, 1)


# speedup vs baseline: 1.0003x; 1.0003x over previous
import jax
import jax.numpy as jnp
from jax.experimental import pallas as pl

_RR = 640


def _make_kernel(T, D):
    U = _RR // D

    def _add_kernel(x_ref, pe_ref, o_ref):
        i = pl.program_id(0)
        peT = pe_ref[...]
        rep = jnp.broadcast_to(peT[None], (U, D, T)).reshape(_RR, T)
        sub = jax.lax.broadcasted_iota(jnp.int32, (_RR, T), 0)
        lane = jax.lax.broadcasted_iota(jnp.int32, (_RR, T), 1)
        mask = lane == (U * i + sub // D)
        pe_col = jnp.sum(jnp.where(mask, rep, 0.0), axis=1, keepdims=True)
        o_ref[0] = x_ref[0] + pe_col

    return _add_kernel


def kernel(x, pos_emb):
    B, T, D = x.shape
    N = T * D
    G = N // _RR
    xt = x.transpose(1, 2, 0).reshape(G, _RR, B)
    out = pl.pallas_call(
        _make_kernel(T, D),
        grid=(G,),
        in_specs=[
            pl.BlockSpec((1, _RR, B), lambda i: (i, 0, 0)),
            pl.BlockSpec((D, T), lambda i: (0, 0)),
        ],
        out_specs=pl.BlockSpec((1, _RR, B), lambda i: (i, 0, 0)),
        out_shape=jax.ShapeDtypeStruct((G, _RR, B), x.dtype),
    )(xt, pos_emb.T)
    return out.reshape(T, D, B).transpose(2, 0, 1)
